# baseline (speedup 1.0000x reference)
import functools

import jax
import jax.numpy as jnp
from jax import lax
from jax.experimental import pallas as pl
from jax.experimental.pallas import tpu as pltpu
from jax.experimental.pallas import tpu_sc as plsc

BATCH = 4
SEQ = 2048
EMBED = 1024
N = BATCH * SEQ

NUM_CORES = 2
NUM_SUBCORES = 16
NW = NUM_CORES * NUM_SUBCORES
ROWS_PER_W = N // NW
CHUNK = 32
NCHUNK = ROWS_PER_W // CHUNK
LANES = 16
VECS_PER_ROW = EMBED // LANES


def _sc_body(x_hbm, tok_hbm, pos_hbm, out_hbm, idx_v, tok_v, pos_v, gsem):
    wid = lax.axis_index("s") * NUM_CORES + lax.axis_index("c")
    base = wid * ROWS_PER_W
    pbase = (wid % (SEQ // ROWS_PER_W)) * ROWS_PER_W

    pltpu.sync_copy(x_hbm.at[pl.ds(base, ROWS_PER_W)], idx_v)

    for c in range(NCHUNK):
        pltpu.async_copy(
            tok_hbm.at[idx_v.at[pl.ds(c * CHUNK, CHUNK)]], tok_v, gsem
        ).wait()
        pltpu.sync_copy(pos_hbm.at[pl.ds(pbase + c * CHUNK, CHUNK)], pos_v)

        def add_row(r, carry):
            for j in range(VECS_PER_ROW):
                sl = pl.ds(j * LANES, LANES)
                tok_v[r, sl] = tok_v[r, sl] + pos_v[r, sl]
            return carry

        lax.fori_loop(0, CHUNK, add_row, 0)

        pltpu.sync_copy(tok_v, out_hbm.at[pl.ds(base + c * CHUNK, CHUNK)])


@jax.jit
def kernel(x, token_table, pos_table):
    xf = x.reshape(-1).astype(jnp.int32)
    mesh = plsc.VectorSubcoreMesh(
        core_axis_name="c", subcore_axis_name="s",
        num_cores=NUM_CORES, num_subcores=NUM_SUBCORES,
    )
    out_flat = pl.kernel(
        _sc_body,
        out_type=jax.ShapeDtypeStruct((N, EMBED), jnp.float32),
        mesh=mesh,
        scratch_types=[
            pltpu.VMEM((ROWS_PER_W,), jnp.int32),
            pltpu.VMEM((CHUNK, EMBED), jnp.float32),
            pltpu.VMEM((CHUNK, EMBED), jnp.float32),
            pltpu.SemaphoreType.DMA,
        ],
    )(xf, token_table, pos_table)
    return out_flat.reshape(BATCH, SEQ, EMBED)


# speedup vs baseline: 1.2970x; 1.2970x over previous
import jax
import jax.numpy as jnp
from jax import lax
from jax.experimental import pallas as pl
from jax.experimental.pallas import tpu as pltpu
from jax.experimental.pallas import tpu_sc as plsc

BATCH = 4
SEQ = 2048
EMBED = 1024
N = BATCH * SEQ

NUM_CORES = 2
NUM_SUBCORES = 16
NW = NUM_CORES * NUM_SUBCORES
POS_PER_W = SEQ // NW
ROWS_PER_W = POS_PER_W * BATCH
CHUNK = 16
CHUNKS_PER_B = POS_PER_W // CHUNK
NCHUNK = CHUNKS_PER_B * BATCH
LANES = 16
VECS_PER_ROW = EMBED // LANES


def _sc_body(x_hbm, tok_hbm, pos_hbm, out_hbm,
             idx_v, pos_v, tok0, tok1, g0, g1, o0, o1, psem):
    wid = lax.axis_index("s") * NUM_CORES + lax.axis_index("c")
    pos0 = wid * POS_PER_W

    for b in range(BATCH):
        pltpu.sync_copy(x_hbm.at[pl.ds(b * SEQ + pos0, POS_PER_W)],
                        idx_v.at[pl.ds(b * POS_PER_W, POS_PER_W)])

    pos_cp = pltpu.async_copy(pos_hbm.at[pl.ds(pos0, POS_PER_W)], pos_v, psem)

    bufs = (tok0, tok1)
    gsems = (g0, g1)
    osems = (o0, o1)

    def idx_slice(c):
        return idx_v.at[pl.ds(c * CHUNK, CHUNK)]

    def out_slice(c):
        b, sub = c // CHUNKS_PER_B, c % CHUNKS_PER_B
        return out_hbm.at[pl.ds(b * SEQ + pos0 + sub * CHUNK, CHUNK)]

    gathers = [None] * NCHUNK
    scatters = [None] * NCHUNK
    gathers[0] = pltpu.async_copy(tok_hbm.at[idx_slice(0)], bufs[0], gsems[0])
    pos_cp.wait()

    for c in range(NCHUNK):
        k = c & 1
        buf = bufs[k]
        gathers[c].wait()
        if c + 1 < NCHUNK:
            if c >= 1:
                scatters[c - 1].wait()
            gathers[c + 1] = pltpu.async_copy(
                tok_hbm.at[idx_slice(c + 1)], bufs[1 - k], gsems[1 - k])

        prow = (c % CHUNKS_PER_B) * CHUNK

        def add_row(r, carry):
            for j in range(VECS_PER_ROW):
                sl = pl.ds(j * LANES, LANES)
                plsc.addupdate(buf.at[r, sl], pos_v[prow + r, sl])
            return carry

        lax.fori_loop(0, CHUNK, add_row, 0)

        scatters[c] = pltpu.async_copy(buf, out_slice(c), osems[k])

    scatters[NCHUNK - 2].wait()
    scatters[NCHUNK - 1].wait()


@jax.jit
def kernel(x, token_table, pos_table):
    xf = x.reshape(-1).astype(jnp.int32)
    mesh = plsc.VectorSubcoreMesh(
        core_axis_name="c", subcore_axis_name="s",
        num_cores=NUM_CORES, num_subcores=NUM_SUBCORES,
    )
    out_flat = pl.kernel(
        _sc_body,
        out_type=jax.ShapeDtypeStruct((N, EMBED), jnp.float32),
        mesh=mesh,
        scratch_types=[
            pltpu.VMEM((ROWS_PER_W,), jnp.int32),
            pltpu.VMEM((POS_PER_W, EMBED), jnp.float32),
            pltpu.VMEM((CHUNK, EMBED), jnp.float32),
            pltpu.VMEM((CHUNK, EMBED), jnp.float32),
            pltpu.SemaphoreType.DMA,
            pltpu.SemaphoreType.DMA,
            pltpu.SemaphoreType.DMA,
            pltpu.SemaphoreType.DMA,
            pltpu.SemaphoreType.DMA,
        ],
    )(xf, token_table, pos_table)
    return out_flat.reshape(BATCH, SEQ, EMBED)
